# baseline (speedup 1.0000x reference)
import functools

import jax
import jax.numpy as jnp
from jax import lax
from jax.experimental import pallas as pl
from jax.experimental.pallas import tpu as pltpu
from jax.experimental.pallas import tpu_sc as plsc

D = 64
NC, NS = 2, 16
NW = NC * NS
BLK = 128
L = 16


@functools.partial(jax.jit, static_argnums=(2, 3))
def _gather_t(table_p, x_t, seq_len, batch):
    n_blk = batch // NW
    assert n_blk == BLK

    mesh = plsc.VectorSubcoreMesh(
        core_axis_name="c", subcore_axis_name="s",
        num_cores=NC, num_subcores=NS)

    @functools.partial(
        pl.kernel,
        mesh=mesh,
        compiler_params=pltpu.CompilerParams(
            use_tc_tiling_on_sc=True, needs_layout_passes=False),
        out_type=jax.ShapeDtypeStruct((seq_len, D, batch), jnp.float32),
        scratch_types=[
            pltpu.VMEM((seq_len, BLK), jnp.int32),
            pltpu.VMEM((2, BLK, BLK), jnp.float32),
            pltpu.VMEM((2, D, BLK), jnp.float32),
            pltpu.SemaphoreType.DMA,
            pltpu.SemaphoreType.DMA,
        ],
    )
    def body(table_hbm, xt_hbm, out_hbm, idx_v, rows_v, trans_v, gsem, osem):
        wid = lax.axis_index("s") * NC + lax.axis_index("c")
        col0 = wid * BLK

        pltpu.sync_copy(xt_hbm.at[:, pl.ds(col0, BLK)], idx_v)

        def fire_gather(s, b):
            pltpu.async_copy(
                table_hbm.at[idx_v.at[s]], rows_v.at[b], gsem)

        def wait_gather(s, b):
            pltpu.make_async_copy(
                table_hbm.at[idx_v.at[s]], rows_v.at[b], gsem).wait()

        def fire_store(s, b):
            pltpu.async_copy(
                trans_v.at[b], out_hbm.at[s, :, pl.ds(col0, BLK)], osem)

        def wait_store(s, b):
            pltpu.make_async_copy(
                trans_v.at[b], out_hbm.at[s, :, pl.ds(col0, BLK)], osem).wait()

        row_iotas = [
            lax.iota(jnp.int32, L) + jnp.int32(g * L) for g in range(BLK // L)
        ]

        def transpose_block(b):
            for d in range(D):
                col = jnp.full((L,), d, jnp.int32)
                for g in range(BLK // L):
                    v = plsc.load_gather(rows_v.at[b], [row_iotas[g], col])
                    trans_v[b, d, pl.ds(g * L, L)] = v

        fire_gather(0, 0)

        @pl.loop(0, seq_len, step=2)
        def _(s2):
            for b in range(2):
                s = s2 + b
                nb = 1 - b

                @pl.when(s + 1 < seq_len)
                def _():
                    fire_gather(s + 1, nb)

                wait_gather(s, b)

                @pl.when(s >= 2)
                def _():
                    wait_store(s - 2, b)

                transpose_block(b)
                fire_store(s, b)

        wait_store(seq_len - 2, 0)
        wait_store(seq_len - 1, 1)

    return body(table_p, x_t)


def kernel(x, embedding_table, train):
    b, s = x.shape
    table_p = jnp.pad(embedding_table, ((0, 0), (0, BLK - D)))
    out_t = _gather_t(table_p, x.T, s, b)
    return out_t.transpose(2, 0, 1)


# speedup vs baseline: 1.5661x; 1.5661x over previous
import functools

import jax
import jax.numpy as jnp
from jax import lax
from jax.experimental import pallas as pl
from jax.experimental.pallas import tpu as pltpu
from jax.experimental.pallas import tpu_sc as plsc

D = 64
NC, NS = 2, 16
NW = NC * NS
BLK = 128
NBUF = 3


@functools.partial(jax.jit, static_argnums=(2, 3))
def _gather_sb(table_p, x_t, seq_len, batch):
    assert batch // NW == BLK

    mesh = plsc.VectorSubcoreMesh(
        core_axis_name="c", subcore_axis_name="s",
        num_cores=NC, num_subcores=NS)

    @functools.partial(
        pl.kernel,
        mesh=mesh,
        compiler_params=pltpu.CompilerParams(
            use_tc_tiling_on_sc=True, needs_layout_passes=False),
        out_type=jax.ShapeDtypeStruct((seq_len, batch, BLK), jnp.float32),
        scratch_types=[
            pltpu.VMEM((seq_len, BLK), jnp.int32),
            pltpu.VMEM((NBUF, BLK, BLK), jnp.float32),
            pltpu.SemaphoreType.DMA,
            pltpu.SemaphoreType.DMA,
        ],
    )
    def body(table_hbm, xt_hbm, out_hbm, idx_v, rows_v, gsem, osem):
        wid = lax.axis_index("s") * NC + lax.axis_index("c")
        col0 = wid * BLK

        pltpu.sync_copy(xt_hbm.at[:, pl.ds(col0, BLK)], idx_v)

        def fire_gather(s, b):
            pltpu.async_copy(
                table_hbm.at[idx_v.at[s]], rows_v.at[b], gsem)

        def wait_gather(s, b):
            pltpu.make_async_copy(
                table_hbm.at[idx_v.at[s]], rows_v.at[b], gsem).wait()

        def fire_store(s, b):
            pltpu.async_copy(
                rows_v.at[b], out_hbm.at[s, pl.ds(col0, BLK)], osem)

        def wait_store(s, b):
            pltpu.make_async_copy(
                rows_v.at[b], out_hbm.at[s, pl.ds(col0, BLK)], osem).wait()

        fire_gather(0, 0)
        fire_gather(1, 1)

        @pl.loop(0, seq_len)
        def _(s):
            b = lax.rem(s, NBUF)
            fb = lax.rem(s + 2, NBUF)

            @pl.when(s + 2 < seq_len)
            def _():
                @pl.when(s >= 1)
                def _():
                    wait_store(s - 1, fb)

                fire_gather(s + 2, fb)

            wait_gather(s, b)
            fire_store(s, b)

        for last in range(seq_len - NBUF, seq_len):
            wait_store(last, last % NBUF)

    return body(table_p, x_t)


def kernel(x, embedding_table, train):
    b, s = x.shape
    table_p = jnp.pad(embedding_table, ((0, 0), (0, BLK - D)))
    rows = _gather_sb(table_p, x.T, s, b)
    return rows[:, :, :D].transpose(1, 0, 2)


# speedup vs baseline: 1.6343x; 1.0435x over previous
import functools

import jax
import jax.numpy as jnp
from jax import lax
from jax.experimental import pallas as pl
from jax.experimental.pallas import tpu as pltpu
from jax.experimental.pallas import tpu_sc as plsc

D = 64
NC, NS = 2, 16
NW = NC * NS
BLK = 128
L = 16


@functools.partial(jax.jit, static_argnums=(2, 3))
def _gather_t(table_p, x_t, seq_len, batch):
    assert batch // NW == BLK

    mesh = plsc.VectorSubcoreMesh(
        core_axis_name="c", subcore_axis_name="s",
        num_cores=NC, num_subcores=NS)

    @functools.partial(
        pl.kernel,
        mesh=mesh,
        compiler_params=pltpu.CompilerParams(
            use_tc_tiling_on_sc=True, needs_layout_passes=False),
        out_type=jax.ShapeDtypeStruct((seq_len, D, batch), jnp.float32),
        scratch_types=[
            pltpu.VMEM((seq_len, BLK), jnp.int32),
            pltpu.VMEM((2, BLK, BLK), jnp.float32),
            pltpu.VMEM((2, D, BLK), jnp.float32),
            pltpu.SemaphoreType.DMA,
            pltpu.SemaphoreType.DMA,
        ],
    )
    def body(table_hbm, xt_hbm, out_hbm, idx_v, rows_v, trans_v, gsem, osem):
        wid = lax.axis_index("s") * NC + lax.axis_index("c")
        col0 = wid * BLK

        pltpu.sync_copy(xt_hbm.at[:, pl.ds(col0, BLK)], idx_v)

        def fire_gather(s, b):
            pltpu.async_copy(
                table_hbm.at[idx_v.at[s]], rows_v.at[b], gsem)

        def wait_gather(s, b):
            pltpu.make_async_copy(
                table_hbm.at[idx_v.at[s]], rows_v.at[b], gsem).wait()

        def fire_store(s, b):
            pltpu.async_copy(
                trans_v.at[b], out_hbm.at[s, :, pl.ds(col0, BLK)], osem)

        def wait_store(s, b):
            pltpu.make_async_copy(
                trans_v.at[b], out_hbm.at[s, :, pl.ds(col0, BLK)], osem).wait()

        row_iotas = [
            lax.iota(jnp.int32, L) + jnp.int32(g * L) for g in range(BLK // L)
        ]

        def transpose_block(b):
            @plsc.parallel_loop(0, D, unroll=8)
            def _(d):
                col = jnp.full((L,), 0, jnp.int32) + d
                for g in range(BLK // L):
                    v = plsc.load_gather(rows_v.at[b], [row_iotas[g], col])
                    trans_v[b, d, pl.ds(g * L, L)] = v

        fire_gather(0, 0)

        @pl.loop(0, seq_len, step=2)
        def _(s2):
            for b in range(2):
                s = s2 + b
                nb = 1 - b

                @pl.when(s + 1 < seq_len)
                def _():
                    fire_gather(s + 1, nb)

                wait_gather(s, b)

                @pl.when(s >= 2)
                def _():
                    wait_store(s - 2, b)

                transpose_block(b)
                fire_store(s, b)

        wait_store(seq_len - 2, 0)
        wait_store(seq_len - 1, 1)

    return body(table_p, x_t)


def kernel(x, embedding_table, train):
    b, s = x.shape
    table_p = jnp.pad(embedding_table, ((0, 0), (0, BLK - D)))
    out_t = _gather_t(table_p, x.T, s, b)
    return out_t.transpose(2, 0, 1)


# speedup vs baseline: 2.4979x; 1.5285x over previous
import functools

import jax
import jax.numpy as jnp
from jax import lax
from jax.experimental import pallas as pl
from jax.experimental.pallas import tpu as pltpu
from jax.experimental.pallas import tpu_sc as plsc

D = 64
NC, NS = 2, 16
NW = NC * NS
BLK = 128
L = 16


@functools.partial(jax.jit, static_argnums=(2, 3))
def _gather_t(table_p, x_t, seq_len, batch):
    assert batch // NW == BLK

    mesh = plsc.VectorSubcoreMesh(
        core_axis_name="c", subcore_axis_name="s",
        num_cores=NC, num_subcores=NS)

    @functools.partial(
        pl.kernel,
        mesh=mesh,
        compiler_params=pltpu.CompilerParams(
            use_tc_tiling_on_sc=True, needs_layout_passes=False),
        out_type=jax.ShapeDtypeStruct((seq_len, D, batch), jnp.float32),
        scratch_types=[
            pltpu.VMEM((seq_len, BLK), jnp.int32),
            pltpu.VMEM((2, BLK, BLK), jnp.float32),
            pltpu.VMEM((2, D, BLK), jnp.float32),
            pltpu.SemaphoreType.DMA,
            pltpu.SemaphoreType.DMA,
        ],
    )
    def body(table_hbm, xt_hbm, out_hbm, idx_v, rows_v, trans_v, gsem, osem):
        wid = lax.axis_index("s") * NC + lax.axis_index("c")
        col0 = wid * BLK

        pltpu.sync_copy(xt_hbm.at[:, pl.ds(col0, BLK)], idx_v)

        def fire_gather(s, b):
            pltpu.async_copy(
                table_hbm.at[idx_v.at[s]], rows_v.at[b], gsem)

        def wait_gather(s, b):
            pltpu.make_async_copy(
                table_hbm.at[idx_v.at[s]], rows_v.at[b], gsem).wait()

        def fire_store(s, b):
            pltpu.async_copy(
                trans_v.at[b], out_hbm.at[s, :, pl.ds(col0, BLK)], osem)

        def wait_store(s, b):
            pltpu.make_async_copy(
                trans_v.at[b], out_hbm.at[s, :, pl.ds(col0, BLK)], osem).wait()

        riota = lax.iota(jnp.int32, L)
        perms = [lax.rem(riota + jnp.int32(k), jnp.int32(L)) for k in range(L)]

        def transpose_block(b):
            @plsc.parallel_loop(0, (D // L) * (BLK // L), unroll=2)
            def _(t):
                di = lax.div(t, BLK // L) * L
                ri = lax.rem(t, BLK // L) * L
                rvec = riota + ri
                for k in range(L):
                    cvec = perms[k] + di
                    v = plsc.load_gather(rows_v.at[b], [rvec, cvec])
                    plsc.store_scatter(trans_v.at[b], [cvec, rvec], v)

        fire_gather(0, 0)

        @pl.loop(0, seq_len, step=2)
        def _(s2):
            for b in range(2):
                s = s2 + b
                nb = 1 - b

                @pl.when(s + 1 < seq_len)
                def _():
                    fire_gather(s + 1, nb)

                wait_gather(s, b)

                @pl.when(s >= 2)
                def _():
                    wait_store(s - 2, b)

                transpose_block(b)
                fire_store(s, b)

        wait_store(seq_len - 2, 0)
        wait_store(seq_len - 1, 1)

    return body(table_p, x_t)


def kernel(x, embedding_table, train):
    b, s = x.shape
    table_p = jnp.pad(embedding_table, ((0, 0), (0, BLK - D)))
    out_t = _gather_t(table_p, x.T, s, b)
    return out_t.transpose(2, 0, 1)
